# baseline (speedup 1.0000x reference)
import jax
import jax.numpy as jnp
from jax import lax
from jax.experimental import pallas as pl
from jax.experimental.pallas import tpu as pltpu

B, S, D = 4, 8192, 1024
CB = 256
NJX = S // CB
NJ = NJX + 1


def _body(idx_ref, x_ref, w_ref, o1_ref, o2_ref, carry_ref):
    j = pl.program_id(0)
    b = pl.program_id(1)

    for kb in range(B):
        @pl.when(b == kb)
        def _(kb=kb):
            @pl.when(j == 0)
            def _():
                sub = idx_ref[kb, 0] % 8
                wrows = lax.broadcasted_iota(jnp.int32, (8, D), 0)
                picked = jnp.where(wrows == sub, w_ref[...], 0.0)
                carry_ref[kb:kb + 1, :] = jnp.sum(picked, axis=0,
                                                  keepdims=True)

            blk = x_ref[0]
            shifted = pltpu.roll(blk, 1, 0)
            o1_ref[:, kb, :] = shifted
            o2_ref[:, kb, :] = shifted
            first = carry_ref[kb:kb + 1, :]
            o1_ref[0:1, kb, :] = first
            o2_ref[0:1, kb, :] = first
            carry_ref[kb:kb + 1, :] = blk[CB - 1:CB, :]


def kernel(x, tissue_vector, registry_weight):
    out_sd = jax.ShapeDtypeStruct((S + 1, B, D), jnp.float32)
    grid_spec = pltpu.PrefetchScalarGridSpec(
        num_scalar_prefetch=1,
        grid=(NJ, B),
        in_specs=[
            pl.BlockSpec((1, CB, D),
                         lambda j, b, idx: (b, jnp.minimum(j, NJX - 1), 0)),
            pl.BlockSpec((8, D), lambda j, b, idx: (idx[b, 0] // 8, 0)),
        ],
        out_specs=[
            pl.BlockSpec((CB, B, D), lambda j, b, idx: (j, 0, 0)),
            pl.BlockSpec((CB, B, D), lambda j, b, idx: (j, 0, 0)),
        ],
        scratch_shapes=[pltpu.VMEM((B, D), jnp.float32)],
    )
    o1t, o2t = pl.pallas_call(
        _body,
        grid_spec=grid_spec,
        out_shape=[out_sd, out_sd],
    )(tissue_vector, x, registry_weight)
    return (jnp.transpose(o1t, (1, 0, 2)), jnp.transpose(o2t, (1, 0, 2)))


# speedup vs baseline: 1.1546x; 1.1546x over previous
import jax
import jax.numpy as jnp
from jax import lax
from jax.experimental import pallas as pl
from jax.experimental.pallas import tpu as pltpu

B, S, D = 4, 8192, 1024
CB = 512
NJX = S // CB
NJ = NJX + 1


def _body(idx_ref, x_ref, w_ref, o1_ref, o2_ref, carry_ref):
    j = pl.program_id(0)
    b = pl.program_id(1)

    for kb in range(B):
        @pl.when(b == kb)
        def _(kb=kb):
            @pl.when(j == 0)
            def _():
                sub = idx_ref[kb, 0] % 8
                wrows = lax.broadcasted_iota(jnp.int32, (8, D), 0)
                picked = jnp.where(wrows == sub, w_ref[...], 0.0)
                carry_ref[kb:kb + 1, :] = jnp.sum(picked, axis=0,
                                                  keepdims=True)

            blk = x_ref[0]
            shifted = pltpu.roll(blk, 1, 0)
            o1_ref[:, kb, :] = shifted
            o2_ref[:, kb, :] = shifted
            first = carry_ref[kb:kb + 1, :]
            o1_ref[0:1, kb, :] = first
            o2_ref[0:1, kb, :] = first
            carry_ref[kb:kb + 1, :] = blk[CB - 1:CB, :]


def kernel(x, tissue_vector, registry_weight):
    out_sd = jax.ShapeDtypeStruct((S + 1, B, D), jnp.float32)
    grid_spec = pltpu.PrefetchScalarGridSpec(
        num_scalar_prefetch=1,
        grid=(NJ, B),
        in_specs=[
            pl.BlockSpec((1, CB, D),
                         lambda j, b, idx: (b, jnp.minimum(j, NJX - 1), 0)),
            pl.BlockSpec((8, D), lambda j, b, idx: (idx[b, 0] // 8, 0)),
        ],
        out_specs=[
            pl.BlockSpec((CB, B, D), lambda j, b, idx: (j, 0, 0)),
            pl.BlockSpec((CB, B, D), lambda j, b, idx: (j, 0, 0)),
        ],
        scratch_shapes=[pltpu.VMEM((B, D), jnp.float32)],
    )
    o1t, o2t = pl.pallas_call(
        _body,
        grid_spec=grid_spec,
        out_shape=[out_sd, out_sd],
    )(tissue_vector, x, registry_weight)
    return (jnp.transpose(o1t, (1, 0, 2)), jnp.transpose(o2t, (1, 0, 2)))
